# baseline (speedup 1.0000x reference)
import functools

import jax
import jax.numpy as jnp
from jax.experimental import pallas as pl


def _chamfer_body(num_i, p_ref, g_ref, d1_ref, d2_ref):
    i = pl.program_id(1)
    p = p_ref[0]
    g = g_ref[0]
    sq1 = jnp.sum(p * p, axis=0)
    sq2 = jnp.sum(g * g, axis=0)
    gm2 = -2.0 * g
    one_p = jnp.ones((1, p.shape[1]), jnp.float32)
    one_g = jnp.ones((1, g.shape[1]), jnp.float32)
    lhs_t = jnp.concatenate([p, one_p], axis=0)
    rhs_t = jnp.concatenate([gm2, sq2[None, :]], axis=0)
    lhs_u = jnp.concatenate([p, sq1[None, :]], axis=0)
    rhs_u = jnp.concatenate([gm2, one_g], axis=0)
    dims = (((0,), (0,)), ((), ()))
    t = jax.lax.dot_general(lhs_t, rhs_t, dims, preferred_element_type=jnp.float32)
    u = jax.lax.dot_general(lhs_u, rhs_u, dims, preferred_element_type=jnp.float32)
    d1_ref[0, 0, :] = jnp.min(t, axis=1) + sq1
    colmin = jnp.min(u, axis=0)

    @pl.when(i == 0)
    def _():
        d2_ref[0, 0, :] = colmin

    @pl.when(i > 0)
    def _():
        d2_ref[0, 0, :] = jnp.minimum(d2_ref[0, 0, :], colmin)

    @pl.when(i == num_i - 1)
    def _():
        d2_ref[0, 0, :] = d2_ref[0, 0, :] + sq2


@functools.partial(jax.jit, static_argnames=("interpret",))
def kernel(pred, gt, interpret=False):
    B, N, _ = pred.shape
    M = gt.shape[1]
    predT = jnp.swapaxes(pred, 1, 2)
    gtT = jnp.swapaxes(gt, 1, 2)
    TN = 1024
    num_i = N // TN
    dist1, dist2 = pl.pallas_call(
        functools.partial(_chamfer_body, num_i),
        grid=(B, num_i),
        in_specs=[
            pl.BlockSpec((1, 3, TN), lambda b, i: (b, 0, i)),
            pl.BlockSpec((1, 3, M), lambda b, i: (b, 0, 0)),
        ],
        out_specs=[
            pl.BlockSpec((1, 1, TN), lambda b, i: (b, 0, i)),
            pl.BlockSpec((1, 1, M), lambda b, i: (b, 0, 0)),
        ],
        out_shape=[
            jax.ShapeDtypeStruct((B, 1, N), jnp.float32),
            jax.ShapeDtypeStruct((B, 1, M), jnp.float32),
        ],
        interpret=interpret,
    )(predT, gtT)
    return jnp.mean(dist1) + jnp.mean(dist2)


# speedup vs baseline: 1.1417x; 1.1417x over previous
import functools

import jax
import jax.numpy as jnp
from jax.experimental import pallas as pl


def _chamfer_body(num_i, p_ref, g_ref, d1_ref, d2_ref):
    i = pl.program_id(1)
    p = p_ref[0]
    g = g_ref[0]
    sq1 = jnp.sum(p * p, axis=0)
    sq2 = jnp.sum(g * g, axis=0)
    pm2 = -2.0 * p
    dims = (((0,), (0,)), ((), ()))
    inner2 = jax.lax.dot_general(
        pm2, g, dims, preferred_element_type=jnp.float32
    )
    t = inner2 + sq2[None, :]
    u = t + sq1[:, None]
    d1_ref[0, 0, :] = jnp.min(t, axis=1) + sq1
    colmin = jnp.min(u, axis=0)

    @pl.when(i == 0)
    def _():
        d2_ref[0, 0, :] = colmin

    @pl.when(i > 0)
    def _():
        d2_ref[0, 0, :] = jnp.minimum(d2_ref[0, 0, :], colmin)


@functools.partial(jax.jit, static_argnames=("interpret",))
def kernel(pred, gt, interpret=False):
    B, N, _ = pred.shape
    M = gt.shape[1]
    predT = jnp.swapaxes(pred, 1, 2)
    gtT = jnp.swapaxes(gt, 1, 2)
    TN = 1024
    num_i = N // TN
    dist1, dist2 = pl.pallas_call(
        functools.partial(_chamfer_body, num_i),
        grid=(B, num_i),
        in_specs=[
            pl.BlockSpec((1, 3, TN), lambda b, i: (b, 0, i)),
            pl.BlockSpec((1, 3, M), lambda b, i: (b, 0, 0)),
        ],
        out_specs=[
            pl.BlockSpec((1, 1, TN), lambda b, i: (b, 0, i)),
            pl.BlockSpec((1, 1, M), lambda b, i: (b, 0, 0)),
        ],
        out_shape=[
            jax.ShapeDtypeStruct((B, 1, N), jnp.float32),
            jax.ShapeDtypeStruct((B, 1, M), jnp.float32),
        ],
        interpret=interpret,
    )(predT, gtT)
    return jnp.mean(dist1) + jnp.mean(dist2)
